# baseline (speedup 1.0000x reference)
import functools

import jax
import jax.numpy as jnp
from jax import lax
from jax.experimental import pallas as pl
from jax.experimental.pallas import tpu as pltpu, tpu_sc as plsc

NC = 2
NS = 16
NW = NC * NS

MAXLEN = 200
EMBED = 64
CB = 128


def _tok_pos_kernel(x_hbm, pos_hbm, tok_hbm, out_hbm,
                    idx_v, gbuf0, gbuf1, sbuf0, sbuf1, pos_v,
                    gsem0, gsem1, osem0, osem1):
    wid = lax.axis_index("s") * NC + lax.axis_index("c")

    pltpu.sync_copy(x_hbm.at[:, wid], idx_v)
    pltpu.sync_copy(pos_hbm, pos_v)

    gbufs = (gbuf0, gbuf1)
    sbufs = (sbuf0, sbuf1)
    gsem = (gsem0, gsem1)
    osem = (osem0, osem1)

    lanes = lax.iota(jnp.int32, 16)
    e_ev = [32 * j + 2 * lanes for j in range(EMBED // 32)]
    e_od = [32 * j + 2 * lanes + 1 for j in range(EMBED // 32)]
    ehi_ev = [e // 8 for e in e_ev]
    elo_ev = [e % 8 for e in e_ev]
    ehi_od = [e // 8 for e in e_od]
    elo_od = [e % 8 for e in e_od]
    himask = jnp.full((16,), -65536, jnp.int32)

    def fire_gather(l, buf, sem):
        return pltpu.async_copy(
            tok_hbm.at[idx_v.at[l // 8, l % 8]], buf, sem)

    def transpose_add_flush(l, gbuf, sbuf, sem):
        lv = jnp.full((16,), l, jnp.int32)
        p_ev = [plsc.load_gather(pos_v, [lv, e]) for e in e_ev]
        p_od = [plsc.load_gather(pos_v, [lv, e]) for e in e_od]
        @plsc.parallel_loop(0, CB, unroll=4)
        def row(t):
            tv = jnp.full((16,), t, jnp.int32)
            for j in range(EMBED // 32):
                packed = gbuf[t, pl.ds(j * 16, 16)]
                ev = plsc.bitcast(packed << 16, jnp.float32) + p_ev[j]
                od = plsc.bitcast(packed & himask, jnp.float32) + p_od[j]
                plsc.store_scatter(sbuf, [ehi_ev[j], elo_ev[j], tv], ev)
                plsc.store_scatter(sbuf, [ehi_od[j], elo_od[j], tv], od)
        pltpu.async_copy(sbuf, out_hbm.at[l, :, wid], sem)

    def half(cc, par):
        l = cc * 2 + par

        @pl.when(cc >= 1)
        def _():
            pltpu.make_async_copy(
                sbufs[par], out_hbm.at[0, :, wid], osem[par]).wait()

        g = fire_gather(l, gbufs[par], gsem[par])

        @pl.when(l >= 1)
        def _():
            transpose_add_flush(l - 1, gbufs[1 - par], sbufs[1 - par],
                                osem[1 - par])

        g.wait()

    def body(cc, _):
        half(cc, 0)
        half(cc, 1)
        return ()

    lax.fori_loop(0, MAXLEN // 2, body, ())

    last = MAXLEN - 1
    transpose_add_flush(last, gbufs[last % 2], sbufs[last % 2], osem[last % 2])
    for par in range(2):
        pltpu.make_async_copy(
            sbufs[par], out_hbm.at[0, :, wid], osem[par]).wait()


def kernel(x, tok_table, pos_table):
    B, L = x.shape
    V, E = tok_table.shape
    assert E == EMBED and L == MAXLEN and B == NW * CB

    x4 = x.T.reshape(L // 8, 8, B // 128, 128).transpose(0, 2, 1, 3)
    tok_i = lax.bitcast_convert_type(
        tok_table.astype(jnp.bfloat16).reshape(V, E // 2, 2), jnp.int32)

    fn = pl.kernel(
        _tok_pos_kernel,
        out_type=jax.ShapeDtypeStruct((L, E // 8, B // 128, 8, 128),
                                      jnp.float32),
        mesh=plsc.VectorSubcoreMesh(core_axis_name="c", subcore_axis_name="s"),
        scratch_types=[
            pltpu.VMEM((L // 8, 8, CB), jnp.int32),
            pltpu.VMEM((CB, EMBED // 2), jnp.int32),
            pltpu.VMEM((CB, EMBED // 2), jnp.int32),
            pltpu.VMEM((E // 8, 8, CB), jnp.float32),
            pltpu.VMEM((E // 8, 8, CB), jnp.float32),
            pltpu.VMEM((L, EMBED), jnp.float32),
            pltpu.SemaphoreType.DMA,
            pltpu.SemaphoreType.DMA,
            pltpu.SemaphoreType.DMA,
            pltpu.SemaphoreType.DMA,
        ],
        compiler_params=pltpu.CompilerParams(use_tc_tiling_on_sc=False,
                                             needs_layout_passes=False),
    )
    out5 = fn(x4, pos_table, tok_i)
    return out5.transpose(2, 4, 0, 1, 3).reshape(B, L, E)


# speedup vs baseline: 1.8575x; 1.8575x over previous
import functools

import jax
import jax.numpy as jnp
from jax import lax
from jax.experimental import pallas as pl
from jax.experimental.pallas import tpu as pltpu, tpu_sc as plsc

NC = 2
NS = 16
NW = NC * NS

MAXLEN = 200
EMBED = 64
CB = 128


def _tok_pos_kernel(x_hbm, pos_hbm, tok_hbm, out_hbm,
                    idx_v, gbuf0, gbuf1, sbuf0, sbuf1, pos_v,
                    gsem0, gsem1, osem0, osem1):
    wid = lax.axis_index("s") * NC + lax.axis_index("c")

    pltpu.sync_copy(x_hbm.at[:, wid], idx_v)
    pltpu.sync_copy(pos_hbm, pos_v)

    gbufs = (gbuf0, gbuf1)
    sbufs = (sbuf0, sbuf1)
    gsem = (gsem0, gsem1)
    osem = (osem0, osem1)

    ehi = [lax.iota(jnp.int32, 16) // 8 + 2 * j for j in range(EMBED // 16)]
    elo = [lax.iota(jnp.int32, 16) % 8 for j in range(EMBED // 16)]

    def fire_gather(l, buf, sem):
        return pltpu.async_copy(
            tok_hbm.at[idx_v.at[l // 8, l % 8]], buf, sem)

    def transpose_add_flush(l, gbuf, sbuf, sem):
        pvecs = [pos_v[l, pl.ds(j * 16, 16)] for j in range(EMBED // 16)]
        @plsc.parallel_loop(0, CB, unroll=4)
        def row(t):
            tv = jnp.full((16,), t, jnp.int32)
            for j in range(EMBED // 16):
                vals = gbuf[t, pl.ds(j * 16, 16)] + pvecs[j]
                plsc.store_scatter(sbuf, [ehi[j], elo[j], tv], vals)
        pltpu.async_copy(sbuf, out_hbm.at[l, :, wid], sem)

    def half(cc, par):
        l = cc * 2 + par

        @pl.when(cc >= 1)
        def _():
            pltpu.make_async_copy(
                sbufs[par], out_hbm.at[0, :, wid], osem[par]).wait()

        g = fire_gather(l, gbufs[par], gsem[par])

        @pl.when(l >= 1)
        def _():
            transpose_add_flush(l - 1, gbufs[1 - par], sbufs[1 - par],
                                osem[1 - par])

        g.wait()

    def body(cc, _):
        half(cc, 0)
        half(cc, 1)
        return ()

    lax.fori_loop(0, MAXLEN // 2, body, ())

    last = MAXLEN - 1
    transpose_add_flush(last, gbufs[last % 2], sbufs[last % 2], osem[last % 2])
    for par in range(2):
        pltpu.make_async_copy(
            sbufs[par], out_hbm.at[0, :, wid], osem[par]).wait()


def kernel(x, tok_table, pos_table):
    B, L = x.shape
    V, E = tok_table.shape
    assert E == EMBED and L == MAXLEN and B == NW * CB

    x4 = x.T.reshape(L // 8, 8, B // 128, 128).transpose(0, 2, 1, 3)

    fn = pl.kernel(
        _tok_pos_kernel,
        out_type=jax.ShapeDtypeStruct((L, E // 8, B // 128, 8, 128),
                                      jnp.float32),
        mesh=plsc.VectorSubcoreMesh(core_axis_name="c", subcore_axis_name="s"),
        scratch_types=[
            pltpu.VMEM((L // 8, 8, CB), jnp.int32),
            pltpu.VMEM((CB, EMBED), jnp.float32),
            pltpu.VMEM((CB, EMBED), jnp.float32),
            pltpu.VMEM((E // 8, 8, CB), jnp.float32),
            pltpu.VMEM((E // 8, 8, CB), jnp.float32),
            pltpu.VMEM((L, EMBED), jnp.float32),
            pltpu.SemaphoreType.DMA,
            pltpu.SemaphoreType.DMA,
            pltpu.SemaphoreType.DMA,
            pltpu.SemaphoreType.DMA,
        ],
        compiler_params=pltpu.CompilerParams(use_tc_tiling_on_sc=False,
                                             needs_layout_passes=False),
    )
    out5 = fn(x4, pos_table, tok_table)
    return out5.transpose(2, 4, 0, 1, 3).reshape(B, L, E)
